# baseline (speedup 1.0000x reference)
import jax
import jax.numpy as jnp
from jax import lax
from jax.experimental import pallas as pl
from jax.experimental.pallas import tpu as pltpu
from jax.experimental.pallas import tpu_sc as plsc

BATCH = 16384
EMBED_DIM = 32
LANES = 16
NUM_CORES = 2
NUM_SUBCORES = 16
NUM_WORKERS = NUM_CORES * NUM_SUBCORES
B_PER_W = BATCH // NUM_WORKERS
CHUNK = 128
NCHUNKS = B_PER_W // CHUNK
GPC = CHUNK // LANES

NROWS = 1000000
WBLK = 2048
QBLK = WBLK // 4
PACK_GRID = (NROWS + WBLK - 1) // WBLK
NLINES = PACK_GRID * QBLK


def _pack_body(in_ref, out_ref):
    stacked = jnp.concatenate(
        [in_ref[:, q * QBLK:(q + 1) * QBLK] for q in range(4)], axis=0)
    eye = jnp.eye(128, dtype=jnp.float32)
    out_ref[...] = lax.dot_general(
        stacked, eye, (((0,), (0,)), ((), ())),
        preferred_element_type=jnp.float32)


_pack = pl.pallas_call(
    _pack_body,
    grid=(PACK_GRID,),
    compiler_params=pltpu.CompilerParams(
        fuse_transposed_lhs_in_matmul=True),
    in_specs=[pl.BlockSpec((EMBED_DIM, WBLK), lambda i: (0, i))],
    out_specs=pl.BlockSpec((QBLK, 128), lambda i: (i, 0)),
    out_shape=jax.ShapeDtypeStruct((NLINES, 128), jnp.float32),
)


def _gmf_body(users_hbm, items_hbm, utab_hbm, itab_hbm, params_hbm,
              out_hbm, uidx_v, iidx_v, uq_v, iq_v, urows_v, irows_v, wv,
              outv, sem_u, sem_i):
    wid = lax.axis_index("s") * NUM_CORES + lax.axis_index("c")
    base = wid * B_PER_W

    for j in range(NCHUNKS):
        pltpu.sync_copy(users_hbm.at[pl.ds(base + j * CHUNK, CHUNK)],
                        uidx_v.at[j])
        pltpu.sync_copy(items_hbm.at[pl.ds(base + j * CHUNK, CHUNK)],
                        iidx_v.at[j])
    pltpu.sync_copy(params_hbm, wv)

    for j in range(NCHUNKS):
        def lineids(k, carry, j=j):
            sl = pl.ds(k * LANES, LANES)
            for idx_ref, q_ref in ((uidx_v, uq_v), (iidx_v, iq_v)):
                i = idx_ref.at[j][sl]
                q_ref.at[j][sl] = (
                    lax.shift_left(lax.shift_right_logical(i, 11), 9)
                    | (i & 511))
            return carry

        lax.fori_loop(0, CHUNK // LANES, lineids, 0)

    w_lo = wv[pl.ds(0, LANES)]
    w_hi = wv[pl.ds(LANES, LANES)]
    bias = wv[pl.ds(2 * LANES, LANES)][0]
    lane = lax.iota(jnp.int32, LANES)

    for j in range(NCHUNKS):
        cu = pltpu.async_copy(utab_hbm.at[uq_v.at[j]], urows_v, sem_u)
        ci = pltpu.async_copy(itab_hbm.at[iq_v.at[j]], irows_v, sem_i)
        cu.wait()
        ci.wait()

        def group(g, carry, j=j):
            rowids = g * LANES + lane
            sl = pl.ds(g * LANES, LANES)
            uo = (lax.shift_right_logical(uidx_v.at[j][sl], 9) & 3)
            io = (lax.shift_right_logical(iidx_v.at[j][sl], 9) & 3)
            uo = uo * EMBED_DIM
            io = io * EMBED_DIM
            acc = jnp.full((LANES,), bias, dtype=jnp.float32)
            for d in range(EMBED_DIM):
                uc = plsc.load_gather(urows_v, [rowids, uo + d])
                ic = plsc.load_gather(irows_v, [rowids, io + d])
                wd = w_lo[d] if d < LANES else w_hi[d - LANES]
                acc = acc + uc * ic * wd
            outv[pl.ds(j * CHUNK + g * LANES, LANES)] = acc
            return carry

        lax.fori_loop(0, GPC, group, 0)

    pltpu.sync_copy(outv, out_hbm.at[pl.ds(base, B_PER_W)])


@jax.jit
def _gmf(users, items, user_table, item_table, params):
    utab = _pack(user_table.T)
    itab = _pack(item_table.T)
    mesh = plsc.VectorSubcoreMesh(core_axis_name="c", subcore_axis_name="s")
    return pl.kernel(
        _gmf_body,
        out_type=jax.ShapeDtypeStruct((BATCH,), jnp.float32),
        mesh=mesh,
        compiler_params=pltpu.CompilerParams(needs_layout_passes=False),
        scratch_types=[
            pltpu.VMEM((NCHUNKS, CHUNK), jnp.int32),
            pltpu.VMEM((NCHUNKS, CHUNK), jnp.int32),
            pltpu.VMEM((NCHUNKS, CHUNK), jnp.int32),
            pltpu.VMEM((NCHUNKS, CHUNK), jnp.int32),
            pltpu.VMEM((CHUNK, 128), jnp.float32),
            pltpu.VMEM((CHUNK, 128), jnp.float32),
            pltpu.VMEM((48,), jnp.float32),
            pltpu.VMEM((B_PER_W,), jnp.float32),
            pltpu.SemaphoreType.DMA,
            pltpu.SemaphoreType.DMA,
        ],
    )(users, items, utab, itab, params)


def kernel(users, items, user_table, item_table, W, b):
    params = jnp.zeros((48,), jnp.float32)
    params = params.at[:EMBED_DIM].set(W.reshape(-1))
    params = params.at[EMBED_DIM:EMBED_DIM + 1].set(b)
    return _gmf(users, items, user_table, item_table, params)


# speedup vs baseline: 2.9540x; 2.9540x over previous
import jax
import jax.numpy as jnp
from jax import lax
from jax.experimental import pallas as pl
from jax.experimental.pallas import tpu as pltpu
from jax.experimental.pallas import tpu_sc as plsc

BATCH = 16384
EMBED_DIM = 32
LANES = 16
NUM_CORES = 2
NUM_SUBCORES = 16
NUM_WORKERS = NUM_CORES * NUM_SUBCORES
B_PER_W = BATCH // NUM_WORKERS
CHUNK = 128
NCHUNKS = B_PER_W // CHUNK
GPC = CHUNK // LANES

NROWS = 1000000
WBLK = 16384
SUBBLK = 2048
QBLK = SUBBLK // 4
PACK_GRID = (NROWS + WBLK - 1) // WBLK
NLINES = PACK_GRID * (WBLK // 4)


def _pack_body(in_ref, out_ref):
    eye = jnp.eye(128, dtype=jnp.float32)
    for sb in range(WBLK // SUBBLK):
        lo = sb * SUBBLK
        stacked = jnp.concatenate(
            [in_ref[:, lo + q * QBLK:lo + (q + 1) * QBLK] for q in range(4)],
            axis=0)
        out_ref[sb * QBLK:(sb + 1) * QBLK, :] = lax.dot_general(
            stacked, eye, (((0,), (0,)), ((), ())),
            preferred_element_type=jnp.float32)


_pack = pl.pallas_call(
    _pack_body,
    grid=(PACK_GRID,),
    compiler_params=pltpu.CompilerParams(
        fuse_transposed_lhs_in_matmul=True),
    in_specs=[pl.BlockSpec((EMBED_DIM, WBLK), lambda i: (0, i))],
    out_specs=pl.BlockSpec((WBLK // 4, 128), lambda i: (i, 0)),
    out_shape=jax.ShapeDtypeStruct((NLINES, 128), jnp.float32),
)


def _gmf_body(users_hbm, items_hbm, utab_hbm, itab_hbm, params_hbm,
              out_hbm, uidx_v, iidx_v, uq_v, iq_v, urows_v, irows_v, wv,
              outv, sem_u, sem_i):
    wid = lax.axis_index("s") * NUM_CORES + lax.axis_index("c")
    base = wid * B_PER_W

    for j in range(NCHUNKS):
        pltpu.sync_copy(users_hbm.at[pl.ds(base + j * CHUNK, CHUNK)],
                        uidx_v.at[j])
        pltpu.sync_copy(items_hbm.at[pl.ds(base + j * CHUNK, CHUNK)],
                        iidx_v.at[j])
    pltpu.sync_copy(params_hbm, wv)

    for j in range(NCHUNKS):
        def lineids(k, carry, j=j):
            sl = pl.ds(k * LANES, LANES)
            for idx_ref, q_ref in ((uidx_v, uq_v), (iidx_v, iq_v)):
                i = idx_ref.at[j][sl]
                q_ref.at[j][sl] = (
                    lax.shift_left(lax.shift_right_logical(i, 11), 9)
                    | (i & 511))
            return carry

        lax.fori_loop(0, CHUNK // LANES, lineids, 0)

    w_lo = wv[pl.ds(0, LANES)]
    w_hi = wv[pl.ds(LANES, LANES)]
    bias = wv[pl.ds(2 * LANES, LANES)][0]
    lane = lax.iota(jnp.int32, LANES)

    for j in range(NCHUNKS):
        cu = pltpu.async_copy(utab_hbm.at[uq_v.at[j]], urows_v, sem_u)
        ci = pltpu.async_copy(itab_hbm.at[iq_v.at[j]], irows_v, sem_i)
        cu.wait()
        ci.wait()

        def group(g, carry, j=j):
            rowids = g * LANES + lane
            sl = pl.ds(g * LANES, LANES)
            uo = (lax.shift_right_logical(uidx_v.at[j][sl], 9) & 3)
            io = (lax.shift_right_logical(iidx_v.at[j][sl], 9) & 3)
            uo = uo * EMBED_DIM
            io = io * EMBED_DIM
            acc = jnp.full((LANES,), bias, dtype=jnp.float32)
            for d in range(EMBED_DIM):
                uc = plsc.load_gather(urows_v, [rowids, uo + d])
                ic = plsc.load_gather(irows_v, [rowids, io + d])
                wd = w_lo[d] if d < LANES else w_hi[d - LANES]
                acc = acc + uc * ic * wd
            outv[pl.ds(j * CHUNK + g * LANES, LANES)] = acc
            return carry

        lax.fori_loop(0, GPC, group, 0)

    pltpu.sync_copy(outv, out_hbm.at[pl.ds(base, B_PER_W)])


@jax.jit
def _gmf(users, items, user_table, item_table, params):
    utab = _pack(user_table.T)
    itab = _pack(item_table.T)
    mesh = plsc.VectorSubcoreMesh(core_axis_name="c", subcore_axis_name="s")
    return pl.kernel(
        _gmf_body,
        out_type=jax.ShapeDtypeStruct((BATCH,), jnp.float32),
        mesh=mesh,
        compiler_params=pltpu.CompilerParams(needs_layout_passes=False),
        scratch_types=[
            pltpu.VMEM((NCHUNKS, CHUNK), jnp.int32),
            pltpu.VMEM((NCHUNKS, CHUNK), jnp.int32),
            pltpu.VMEM((NCHUNKS, CHUNK), jnp.int32),
            pltpu.VMEM((NCHUNKS, CHUNK), jnp.int32),
            pltpu.VMEM((CHUNK, 128), jnp.float32),
            pltpu.VMEM((CHUNK, 128), jnp.float32),
            pltpu.VMEM((48,), jnp.float32),
            pltpu.VMEM((B_PER_W,), jnp.float32),
            pltpu.SemaphoreType.DMA,
            pltpu.SemaphoreType.DMA,
        ],
    )(users, items, utab, itab, params)


def kernel(users, items, user_table, item_table, W, b):
    params = jnp.zeros((48,), jnp.float32)
    params = params.at[:EMBED_DIM].set(W.reshape(-1))
    params = params.at[EMBED_DIM:EMBED_DIM + 1].set(b)
    return _gmf(users, items, user_table, item_table, params)


# speedup vs baseline: 3.3572x; 1.1365x over previous
import jax
import jax.numpy as jnp
from jax import lax
from jax.experimental import pallas as pl
from jax.experimental.pallas import tpu as pltpu
from jax.experimental.pallas import tpu_sc as plsc

BATCH = 16384
EMBED_DIM = 32
LANES = 16
NUM_CORES = 2
NUM_SUBCORES = 16
NUM_WORKERS = NUM_CORES * NUM_SUBCORES
B_PER_W = BATCH // NUM_WORKERS
CHUNK = 128
NCHUNKS = B_PER_W // CHUNK
GPC = CHUNK // LANES

NROWS = 1000000
WBLK = 32768
SUBBLK = 2048
QBLK = SUBBLK // 4
PACK_GRID = (NROWS + WBLK - 1) // WBLK
NLINES = PACK_GRID * (WBLK // 4)


def _pack_body(in_ref, out_ref):
    eye = jnp.eye(128, dtype=jnp.float32)
    for sb in range(WBLK // SUBBLK):
        lo = sb * SUBBLK
        stacked = jnp.concatenate(
            [in_ref[:, lo + q * QBLK:lo + (q + 1) * QBLK] for q in range(4)],
            axis=0)
        out_ref[sb * QBLK:(sb + 1) * QBLK, :] = lax.dot_general(
            stacked, eye, (((0,), (0,)), ((), ())),
            preferred_element_type=jnp.float32)


_pack = pl.pallas_call(
    _pack_body,
    grid=(PACK_GRID,),
    compiler_params=pltpu.CompilerParams(
        fuse_transposed_lhs_in_matmul=True),
    in_specs=[pl.BlockSpec((EMBED_DIM, WBLK), lambda i: (0, i))],
    out_specs=pl.BlockSpec((WBLK // 4, 128), lambda i: (i, 0)),
    out_shape=jax.ShapeDtypeStruct((NLINES, 128), jnp.float32),
)


def _gmf_body(users_hbm, items_hbm, utab_hbm, itab_hbm, params_hbm,
              out_hbm, uidx_v, iidx_v, uq_v, iq_v, urows_v, irows_v, wv,
              outv, sem_u, sem_i):
    wid = lax.axis_index("s") * NUM_CORES + lax.axis_index("c")
    base = wid * B_PER_W

    for j in range(NCHUNKS):
        pltpu.sync_copy(users_hbm.at[pl.ds(base + j * CHUNK, CHUNK)],
                        uidx_v.at[j])
        pltpu.sync_copy(items_hbm.at[pl.ds(base + j * CHUNK, CHUNK)],
                        iidx_v.at[j])
    pltpu.sync_copy(params_hbm, wv)

    for j in range(NCHUNKS):
        def lineids(k, carry, j=j):
            sl = pl.ds(k * LANES, LANES)
            for idx_ref, q_ref in ((uidx_v, uq_v), (iidx_v, iq_v)):
                i = idx_ref.at[j][sl]
                q_ref.at[j][sl] = (
                    lax.shift_left(lax.shift_right_logical(i, 11), 9)
                    | (i & 511))
            return carry

        lax.fori_loop(0, CHUNK // LANES, lineids, 0)

    w_lo = wv[pl.ds(0, LANES)]
    w_hi = wv[pl.ds(LANES, LANES)]
    bias = wv[pl.ds(2 * LANES, LANES)][0]
    lane = lax.iota(jnp.int32, LANES)

    for j in range(NCHUNKS):
        cu = pltpu.async_copy(utab_hbm.at[uq_v.at[j]], urows_v, sem_u)
        ci = pltpu.async_copy(itab_hbm.at[iq_v.at[j]], irows_v, sem_i)
        cu.wait()
        ci.wait()

        def group(g, carry, j=j):
            rowids = g * LANES + lane
            sl = pl.ds(g * LANES, LANES)
            uo = (lax.shift_right_logical(uidx_v.at[j][sl], 9) & 3)
            io = (lax.shift_right_logical(iidx_v.at[j][sl], 9) & 3)
            uo = uo * EMBED_DIM
            io = io * EMBED_DIM
            acc = jnp.full((LANES,), bias, dtype=jnp.float32)
            for d in range(EMBED_DIM):
                uc = plsc.load_gather(urows_v, [rowids, uo + d])
                ic = plsc.load_gather(irows_v, [rowids, io + d])
                wd = w_lo[d] if d < LANES else w_hi[d - LANES]
                acc = acc + uc * ic * wd
            outv[pl.ds(j * CHUNK + g * LANES, LANES)] = acc
            return carry

        lax.fori_loop(0, GPC, group, 0)

    pltpu.sync_copy(outv, out_hbm.at[pl.ds(base, B_PER_W)])


@jax.jit
def _gmf(users, items, user_table, item_table, params):
    utab = _pack(user_table.T)
    itab = _pack(item_table.T)
    mesh = plsc.VectorSubcoreMesh(core_axis_name="c", subcore_axis_name="s")
    return pl.kernel(
        _gmf_body,
        out_type=jax.ShapeDtypeStruct((BATCH,), jnp.float32),
        mesh=mesh,
        compiler_params=pltpu.CompilerParams(needs_layout_passes=False),
        scratch_types=[
            pltpu.VMEM((NCHUNKS, CHUNK), jnp.int32),
            pltpu.VMEM((NCHUNKS, CHUNK), jnp.int32),
            pltpu.VMEM((NCHUNKS, CHUNK), jnp.int32),
            pltpu.VMEM((NCHUNKS, CHUNK), jnp.int32),
            pltpu.VMEM((CHUNK, 128), jnp.float32),
            pltpu.VMEM((CHUNK, 128), jnp.float32),
            pltpu.VMEM((48,), jnp.float32),
            pltpu.VMEM((B_PER_W,), jnp.float32),
            pltpu.SemaphoreType.DMA,
            pltpu.SemaphoreType.DMA,
        ],
    )(users, items, utab, itab, params)


def kernel(users, items, user_table, item_table, W, b):
    params = jnp.zeros((48,), jnp.float32)
    params = params.at[:EMBED_DIM].set(W.reshape(-1))
    params = params.at[EMBED_DIM:EMBED_DIM + 1].set(b)
    return _gmf(users, items, user_table, item_table, params)


# speedup vs baseline: 3.5112x; 1.0459x over previous
import jax
import jax.numpy as jnp
from jax import lax
from jax.experimental import pallas as pl
from jax.experimental.pallas import tpu as pltpu
from jax.experimental.pallas import tpu_sc as plsc

BATCH = 16384
EMBED_DIM = 32
LANES = 16
NUM_CORES = 2
NUM_SUBCORES = 16
NUM_WORKERS = NUM_CORES * NUM_SUBCORES
B_PER_W = BATCH // NUM_WORKERS
CHUNK = 128
NCHUNKS = B_PER_W // CHUNK
GPC = CHUNK // LANES

NROWS = 1000000
WBLK = 32768
SUBBLK = 2048
QBLK = SUBBLK // 4
PACK_GRID = (NROWS + WBLK - 1) // WBLK
NLINES = PACK_GRID * (WBLK // 4)


def _pack_body(in_ref, out_ref):
    eye = jnp.eye(128, dtype=jnp.float32)
    for sb in range(WBLK // SUBBLK):
        lo = sb * SUBBLK
        stacked = jnp.concatenate(
            [in_ref[:, lo + q * QBLK:lo + (q + 1) * QBLK] for q in range(4)],
            axis=0)
        out_ref[sb * QBLK:(sb + 1) * QBLK, :] = lax.dot_general(
            stacked, eye, (((0,), (0,)), ((), ())),
            preferred_element_type=jnp.float32)


_pack = pl.pallas_call(
    _pack_body,
    grid=(PACK_GRID,),
    compiler_params=pltpu.CompilerParams(
        fuse_transposed_lhs_in_matmul=True),
    in_specs=[pl.BlockSpec((EMBED_DIM, WBLK), lambda i: (0, i))],
    out_specs=pl.BlockSpec((WBLK // 4, 128), lambda i: (i, 0)),
    out_shape=jax.ShapeDtypeStruct((NLINES, 128), jnp.float32),
)


def _stage1_body(users_hbm, utab_hbm, out_hbm, uidx_v, uq_v, urows_v,
                 ucompT_v, sem_u):
    wid = lax.axis_index("s") * NUM_CORES + lax.axis_index("c")
    base = wid * B_PER_W

    for j in range(NCHUNKS):
        pltpu.sync_copy(users_hbm.at[pl.ds(base + j * CHUNK, CHUNK)],
                        uidx_v.at[j])

    for j in range(NCHUNKS):
        def lineids(k, carry, j=j):
            sl = pl.ds(k * LANES, LANES)
            i = uidx_v.at[j][sl]
            uq_v.at[j][sl] = (
                lax.shift_left(lax.shift_right_logical(i, 11), 9)
                | (i & 511))
            return carry

        lax.fori_loop(0, CHUNK // LANES, lineids, 0)

    lane = lax.iota(jnp.int32, LANES)
    for j in range(NCHUNKS):
        pltpu.async_copy(utab_hbm.at[uq_v.at[j]], urows_v, sem_u).wait()

        def group(g, carry, j=j):
            rowids = g * LANES + lane
            sl = pl.ds(g * LANES, LANES)
            uo = (lax.shift_right_logical(uidx_v.at[j][sl], 9) & 3)
            uo = uo * EMBED_DIM
            for d in range(EMBED_DIM):
                uc = plsc.load_gather(urows_v, [rowids, uo + d])
                ucompT_v[d, pl.ds(j * CHUNK + g * LANES, LANES)] = uc
            return carry

        lax.fori_loop(0, GPC, group, 0)

    pltpu.sync_copy(ucompT_v, out_hbm.at[pl.ds(wid * EMBED_DIM, EMBED_DIM)])


def _stage2_body(items_hbm, itab_hbm, stage_hbm, params_hbm, out_hbm,
                 iidx_v, iq_v, irows_v, ucompT_v, wv, outv, sem_i):
    wid = lax.axis_index("s") * NUM_CORES + lax.axis_index("c")
    base = wid * B_PER_W

    for j in range(NCHUNKS):
        pltpu.sync_copy(items_hbm.at[pl.ds(base + j * CHUNK, CHUNK)],
                        iidx_v.at[j])
    pltpu.sync_copy(params_hbm, wv)
    pltpu.sync_copy(stage_hbm.at[pl.ds(wid * EMBED_DIM, EMBED_DIM)],
                    ucompT_v)

    for j in range(NCHUNKS):
        def lineids(k, carry, j=j):
            sl = pl.ds(k * LANES, LANES)
            i = iidx_v.at[j][sl]
            iq_v.at[j][sl] = (
                lax.shift_left(lax.shift_right_logical(i, 11), 9)
                | (i & 511))
            return carry

        lax.fori_loop(0, CHUNK // LANES, lineids, 0)

    w_lo = wv[pl.ds(0, LANES)]
    w_hi = wv[pl.ds(LANES, LANES)]
    bias = wv[pl.ds(2 * LANES, LANES)][0]
    lane = lax.iota(jnp.int32, LANES)

    for j in range(NCHUNKS):
        pltpu.async_copy(itab_hbm.at[iq_v.at[j]], irows_v, sem_i).wait()

        def group(g, carry, j=j):
            rowids = g * LANES + lane
            sl = pl.ds(g * LANES, LANES)
            io = (lax.shift_right_logical(iidx_v.at[j][sl], 9) & 3)
            io = io * EMBED_DIM
            acc = jnp.full((LANES,), bias, dtype=jnp.float32)
            for d in range(EMBED_DIM):
                ic = plsc.load_gather(irows_v, [rowids, io + d])
                uc = ucompT_v[d, pl.ds(j * CHUNK + g * LANES, LANES)]
                wd = w_lo[d] if d < LANES else w_hi[d - LANES]
                acc = acc + uc * ic * wd
            outv[pl.ds(j * CHUNK + g * LANES, LANES)] = acc
            return carry

        lax.fori_loop(0, GPC, group, 0)

    pltpu.sync_copy(outv, out_hbm.at[pl.ds(base, B_PER_W)])


@jax.jit
def _gmf(users, items, user_table, item_table, params):
    mesh = plsc.VectorSubcoreMesh(core_axis_name="c", subcore_axis_name="s")
    utab = _pack(user_table.T)
    stage = pl.kernel(
        _stage1_body,
        out_type=jax.ShapeDtypeStruct((NUM_WORKERS * EMBED_DIM, B_PER_W),
                                      jnp.float32),
        mesh=mesh,
        compiler_params=pltpu.CompilerParams(needs_layout_passes=False),
        scratch_types=[
            pltpu.VMEM((NCHUNKS, CHUNK), jnp.int32),
            pltpu.VMEM((NCHUNKS, CHUNK), jnp.int32),
            pltpu.VMEM((CHUNK, 128), jnp.float32),
            pltpu.VMEM((EMBED_DIM, B_PER_W), jnp.float32),
            pltpu.SemaphoreType.DMA,
        ],
    )(users, utab)
    itab = _pack(item_table.T)
    return pl.kernel(
        _stage2_body,
        out_type=jax.ShapeDtypeStruct((BATCH,), jnp.float32),
        mesh=mesh,
        compiler_params=pltpu.CompilerParams(needs_layout_passes=False),
        scratch_types=[
            pltpu.VMEM((NCHUNKS, CHUNK), jnp.int32),
            pltpu.VMEM((NCHUNKS, CHUNK), jnp.int32),
            pltpu.VMEM((CHUNK, 128), jnp.float32),
            pltpu.VMEM((EMBED_DIM, B_PER_W), jnp.float32),
            pltpu.VMEM((48,), jnp.float32),
            pltpu.VMEM((B_PER_W,), jnp.float32),
            pltpu.SemaphoreType.DMA,
        ],
    )(items, itab, stage, params)


def kernel(users, items, user_table, item_table, W, b):
    params = jnp.zeros((48,), jnp.float32)
    params = params.at[:EMBED_DIM].set(W.reshape(-1))
    params = params.at[EMBED_DIM:EMBED_DIM + 1].set(b)
    return _gmf(users, items, user_table, item_table, params)


# speedup vs baseline: 3.5490x; 1.0108x over previous
import jax
import jax.numpy as jnp
from jax import lax
from jax.experimental import pallas as pl
from jax.experimental.pallas import tpu as pltpu
from jax.experimental.pallas import tpu_sc as plsc

BATCH = 16384
EMBED_DIM = 32
LANES = 16
NUM_CORES = 2
NUM_SUBCORES = 16
NUM_WORKERS = NUM_CORES * NUM_SUBCORES
B_PER_W = BATCH // NUM_WORKERS
CHUNK = 128
NCHUNKS = B_PER_W // CHUNK
GPC = CHUNK // LANES

NROWS = 1000000
WBLK = 65536
SUBBLK = 2048
QBLK = SUBBLK // 4
PACK_GRID = (NROWS + WBLK - 1) // WBLK
NLINES = PACK_GRID * (WBLK // 4)


def _pack_body(in_ref, out_ref):
    eye = jnp.eye(128, dtype=jnp.float32)
    for sb in range(WBLK // SUBBLK):
        lo = sb * SUBBLK
        stacked = jnp.concatenate(
            [in_ref[:, lo + q * QBLK:lo + (q + 1) * QBLK] for q in range(4)],
            axis=0)
        out_ref[sb * QBLK:(sb + 1) * QBLK, :] = lax.dot_general(
            stacked, eye, (((0,), (0,)), ((), ())),
            preferred_element_type=jnp.float32)


_pack = pl.pallas_call(
    _pack_body,
    grid=(PACK_GRID,),
    compiler_params=pltpu.CompilerParams(
        fuse_transposed_lhs_in_matmul=True),
    in_specs=[pl.BlockSpec((EMBED_DIM, WBLK), lambda i: (0, i))],
    out_specs=pl.BlockSpec((WBLK // 4, 128), lambda i: (i, 0)),
    out_shape=jax.ShapeDtypeStruct((NLINES, 128), jnp.float32),
)


def _stage1_body(users_hbm, utab_hbm, out_hbm, uidx_v, uq_v, urows_v,
                 ucompT_v, sem_u):
    wid = lax.axis_index("s") * NUM_CORES + lax.axis_index("c")
    base = wid * B_PER_W

    for j in range(NCHUNKS):
        pltpu.sync_copy(users_hbm.at[pl.ds(base + j * CHUNK, CHUNK)],
                        uidx_v.at[j])

    for j in range(NCHUNKS):
        def lineids(k, carry, j=j):
            sl = pl.ds(k * LANES, LANES)
            i = uidx_v.at[j][sl]
            uq_v.at[j][sl] = (
                lax.shift_left(lax.shift_right_logical(i, 11), 9)
                | (i & 511))
            return carry

        lax.fori_loop(0, CHUNK // LANES, lineids, 0)

    lane = lax.iota(jnp.int32, LANES)
    for j in range(NCHUNKS):
        pltpu.async_copy(utab_hbm.at[uq_v.at[j]], urows_v, sem_u).wait()

        def group(g, carry, j=j):
            rowids = g * LANES + lane
            sl = pl.ds(g * LANES, LANES)
            uo = (lax.shift_right_logical(uidx_v.at[j][sl], 9) & 3)
            uo = uo * EMBED_DIM
            for d in range(EMBED_DIM):
                uc = plsc.load_gather(urows_v, [rowids, uo + d])
                ucompT_v[d, pl.ds(j * CHUNK + g * LANES, LANES)] = uc
            return carry

        lax.fori_loop(0, GPC, group, 0)

    pltpu.sync_copy(ucompT_v, out_hbm.at[pl.ds(wid * EMBED_DIM, EMBED_DIM)])


def _stage2_body(items_hbm, itab_hbm, stage_hbm, params_hbm, out_hbm,
                 iidx_v, iq_v, irows_v, ucompT_v, wv, outv, sem_i):
    wid = lax.axis_index("s") * NUM_CORES + lax.axis_index("c")
    base = wid * B_PER_W

    for j in range(NCHUNKS):
        pltpu.sync_copy(items_hbm.at[pl.ds(base + j * CHUNK, CHUNK)],
                        iidx_v.at[j])
    pltpu.sync_copy(params_hbm, wv)
    pltpu.sync_copy(stage_hbm.at[pl.ds(wid * EMBED_DIM, EMBED_DIM)],
                    ucompT_v)

    for j in range(NCHUNKS):
        def lineids(k, carry, j=j):
            sl = pl.ds(k * LANES, LANES)
            i = iidx_v.at[j][sl]
            iq_v.at[j][sl] = (
                lax.shift_left(lax.shift_right_logical(i, 11), 9)
                | (i & 511))
            return carry

        lax.fori_loop(0, CHUNK // LANES, lineids, 0)

    w_lo = wv[pl.ds(0, LANES)]
    w_hi = wv[pl.ds(LANES, LANES)]
    bias = wv[pl.ds(2 * LANES, LANES)][0]
    lane = lax.iota(jnp.int32, LANES)

    for j in range(NCHUNKS):
        pltpu.async_copy(itab_hbm.at[iq_v.at[j]], irows_v, sem_i).wait()

        def group(g, carry, j=j):
            rowids = g * LANES + lane
            sl = pl.ds(g * LANES, LANES)
            io = (lax.shift_right_logical(iidx_v.at[j][sl], 9) & 3)
            io = io * EMBED_DIM
            acc = jnp.full((LANES,), bias, dtype=jnp.float32)
            for d in range(EMBED_DIM):
                ic = plsc.load_gather(irows_v, [rowids, io + d])
                uc = ucompT_v[d, pl.ds(j * CHUNK + g * LANES, LANES)]
                wd = w_lo[d] if d < LANES else w_hi[d - LANES]
                acc = acc + uc * ic * wd
            outv[pl.ds(j * CHUNK + g * LANES, LANES)] = acc
            return carry

        lax.fori_loop(0, GPC, group, 0)

    pltpu.sync_copy(outv, out_hbm.at[pl.ds(base, B_PER_W)])


@jax.jit
def _gmf(users, items, user_table, item_table, params):
    mesh = plsc.VectorSubcoreMesh(core_axis_name="c", subcore_axis_name="s")
    utab = _pack(user_table.T)
    stage = pl.kernel(
        _stage1_body,
        out_type=jax.ShapeDtypeStruct((NUM_WORKERS * EMBED_DIM, B_PER_W),
                                      jnp.float32),
        mesh=mesh,
        compiler_params=pltpu.CompilerParams(needs_layout_passes=False),
        scratch_types=[
            pltpu.VMEM((NCHUNKS, CHUNK), jnp.int32),
            pltpu.VMEM((NCHUNKS, CHUNK), jnp.int32),
            pltpu.VMEM((CHUNK, 128), jnp.float32),
            pltpu.VMEM((EMBED_DIM, B_PER_W), jnp.float32),
            pltpu.SemaphoreType.DMA,
        ],
    )(users, utab)
    itab = _pack(item_table.T)
    return pl.kernel(
        _stage2_body,
        out_type=jax.ShapeDtypeStruct((BATCH,), jnp.float32),
        mesh=mesh,
        compiler_params=pltpu.CompilerParams(needs_layout_passes=False),
        scratch_types=[
            pltpu.VMEM((NCHUNKS, CHUNK), jnp.int32),
            pltpu.VMEM((NCHUNKS, CHUNK), jnp.int32),
            pltpu.VMEM((CHUNK, 128), jnp.float32),
            pltpu.VMEM((EMBED_DIM, B_PER_W), jnp.float32),
            pltpu.VMEM((48,), jnp.float32),
            pltpu.VMEM((B_PER_W,), jnp.float32),
            pltpu.SemaphoreType.DMA,
        ],
    )(items, itab, stage, params)


def kernel(users, items, user_table, item_table, W, b):
    params = jnp.zeros((48,), jnp.float32)
    params = params.at[:EMBED_DIM].set(W.reshape(-1))
    params = params.at[EMBED_DIM:EMBED_DIM + 1].set(b)
    return _gmf(users, items, user_table, item_table, params)


# speedup vs baseline: 3.5897x; 1.0115x over previous
import jax
import jax.numpy as jnp
from jax import lax
from jax.experimental import pallas as pl
from jax.experimental.pallas import tpu as pltpu
from jax.experimental.pallas import tpu_sc as plsc

BATCH = 16384
EMBED_DIM = 32
LANES = 16
NUM_CORES = 2
NUM_SUBCORES = 16
NUM_WORKERS = NUM_CORES * NUM_SUBCORES
B_PER_W = BATCH // NUM_WORKERS
CHUNK = 128
NCHUNKS = B_PER_W // CHUNK
GPC = CHUNK // LANES

NROWS = 1000000
WBLK = 65536
SUBBLK = 2048
QBLK = SUBBLK // 4
PACK_GRID = (NROWS + WBLK - 1) // WBLK
NLINES = PACK_GRID * (WBLK // 4)


def _pack_body(in_ref, out_ref):
    eye = jnp.eye(128, dtype=jnp.float32)
    for sb in range(WBLK // SUBBLK):
        lo = sb * SUBBLK
        stacked = jnp.concatenate(
            [in_ref[:, lo + q * QBLK:lo + (q + 1) * QBLK] for q in range(4)],
            axis=0)
        out_ref[sb * QBLK:(sb + 1) * QBLK, :] = lax.dot_general(
            stacked, eye, (((0,), (0,)), ((), ())),
            preferred_element_type=jnp.float32)


_pack = pl.pallas_call(
    _pack_body,
    grid=(PACK_GRID,),
    compiler_params=pltpu.CompilerParams(
        fuse_transposed_lhs_in_matmul=True),
    in_specs=[pl.BlockSpec((EMBED_DIM, WBLK), lambda i: (0, i))],
    out_specs=pl.BlockSpec((WBLK // 4, 128), lambda i: (i, 0)),
    out_shape=jax.ShapeDtypeStruct((NLINES, 128), jnp.float32),
)


def _stage1_body(users_hbm, utab_hbm, out_hbm, uidx_v, uq_v, urows_v,
                 urows_b, ucompT_v, sem_u, sem_b):
    wid = lax.axis_index("s") * NUM_CORES + lax.axis_index("c")
    base = wid * B_PER_W

    for j in range(NCHUNKS):
        pltpu.sync_copy(users_hbm.at[pl.ds(base + j * CHUNK, CHUNK)],
                        uidx_v.at[j])

    for j in range(NCHUNKS):
        def lineids(k, carry, j=j):
            sl = pl.ds(k * LANES, LANES)
            i = uidx_v.at[j][sl]
            uq_v.at[j][sl] = (
                lax.shift_left(lax.shift_right_logical(i, 11), 9)
                | (i & 511))
            return carry

        lax.fori_loop(0, CHUNK // LANES, lineids, 0)

    lane = lax.iota(jnp.int32, LANES)
    bufs = (urows_v, urows_b)
    sems = (sem_u, sem_b)
    copies = [pltpu.async_copy(utab_hbm.at[uq_v.at[0]], bufs[0], sems[0])]
    for j in range(NCHUNKS):
        if j + 1 < NCHUNKS:
            copies.append(pltpu.async_copy(
                utab_hbm.at[uq_v.at[j + 1]], bufs[(j + 1) % 2],
                sems[(j + 1) % 2]))
        copies[j].wait()
        rows_v = bufs[j % 2]

        def group(g, carry, j=j, rows_v=rows_v):
            rowids = g * LANES + lane
            sl = pl.ds(g * LANES, LANES)
            uo = (lax.shift_right_logical(uidx_v.at[j][sl], 9) & 3)
            uo = uo * EMBED_DIM
            for d in range(EMBED_DIM):
                uc = plsc.load_gather(rows_v, [rowids, uo + d])
                ucompT_v[d, pl.ds(j * CHUNK + g * LANES, LANES)] = uc
            return carry

        lax.fori_loop(0, GPC, group, 0)

    pltpu.sync_copy(ucompT_v, out_hbm.at[pl.ds(wid * EMBED_DIM, EMBED_DIM)])


def _stage2_body(items_hbm, itab_hbm, stage_hbm, params_hbm, out_hbm,
                 iidx_v, iq_v, irows_v, irows_b, ucompT_v, wv, outv,
                 sem_i, sem_b):
    wid = lax.axis_index("s") * NUM_CORES + lax.axis_index("c")
    base = wid * B_PER_W

    for j in range(NCHUNKS):
        pltpu.sync_copy(items_hbm.at[pl.ds(base + j * CHUNK, CHUNK)],
                        iidx_v.at[j])
    pltpu.sync_copy(params_hbm, wv)
    pltpu.sync_copy(stage_hbm.at[pl.ds(wid * EMBED_DIM, EMBED_DIM)],
                    ucompT_v)

    for j in range(NCHUNKS):
        def lineids(k, carry, j=j):
            sl = pl.ds(k * LANES, LANES)
            i = iidx_v.at[j][sl]
            iq_v.at[j][sl] = (
                lax.shift_left(lax.shift_right_logical(i, 11), 9)
                | (i & 511))
            return carry

        lax.fori_loop(0, CHUNK // LANES, lineids, 0)

    w_lo = wv[pl.ds(0, LANES)]
    w_hi = wv[pl.ds(LANES, LANES)]
    bias = wv[pl.ds(2 * LANES, LANES)][0]
    lane = lax.iota(jnp.int32, LANES)

    bufs = (irows_v, irows_b)
    sems = (sem_i, sem_b)
    copies = [pltpu.async_copy(itab_hbm.at[iq_v.at[0]], bufs[0], sems[0])]
    for j in range(NCHUNKS):
        if j + 1 < NCHUNKS:
            copies.append(pltpu.async_copy(
                itab_hbm.at[iq_v.at[j + 1]], bufs[(j + 1) % 2],
                sems[(j + 1) % 2]))
        copies[j].wait()
        rows_v = bufs[j % 2]

        def group(g, carry, j=j, rows_v=rows_v):
            rowids = g * LANES + lane
            sl = pl.ds(g * LANES, LANES)
            io = (lax.shift_right_logical(iidx_v.at[j][sl], 9) & 3)
            io = io * EMBED_DIM
            acc = jnp.full((LANES,), bias, dtype=jnp.float32)
            for d in range(EMBED_DIM):
                ic = plsc.load_gather(rows_v, [rowids, io + d])
                uc = ucompT_v[d, pl.ds(j * CHUNK + g * LANES, LANES)]
                wd = w_lo[d] if d < LANES else w_hi[d - LANES]
                acc = acc + uc * ic * wd
            outv[pl.ds(j * CHUNK + g * LANES, LANES)] = acc
            return carry

        lax.fori_loop(0, GPC, group, 0)

    pltpu.sync_copy(outv, out_hbm.at[pl.ds(base, B_PER_W)])


@jax.jit
def _gmf(users, items, user_table, item_table, params):
    mesh = plsc.VectorSubcoreMesh(core_axis_name="c", subcore_axis_name="s")
    utab = _pack(user_table.T)
    stage = pl.kernel(
        _stage1_body,
        out_type=jax.ShapeDtypeStruct((NUM_WORKERS * EMBED_DIM, B_PER_W),
                                      jnp.float32),
        mesh=mesh,
        compiler_params=pltpu.CompilerParams(needs_layout_passes=False),
        scratch_types=[
            pltpu.VMEM((NCHUNKS, CHUNK), jnp.int32),
            pltpu.VMEM((NCHUNKS, CHUNK), jnp.int32),
            pltpu.VMEM((CHUNK, 128), jnp.float32),
            pltpu.VMEM((CHUNK, 128), jnp.float32),
            pltpu.VMEM((EMBED_DIM, B_PER_W), jnp.float32),
            pltpu.SemaphoreType.DMA,
            pltpu.SemaphoreType.DMA,
        ],
    )(users, utab)
    itab = _pack(item_table.T)
    return pl.kernel(
        _stage2_body,
        out_type=jax.ShapeDtypeStruct((BATCH,), jnp.float32),
        mesh=mesh,
        compiler_params=pltpu.CompilerParams(needs_layout_passes=False),
        scratch_types=[
            pltpu.VMEM((NCHUNKS, CHUNK), jnp.int32),
            pltpu.VMEM((NCHUNKS, CHUNK), jnp.int32),
            pltpu.VMEM((CHUNK, 128), jnp.float32),
            pltpu.VMEM((CHUNK, 128), jnp.float32),
            pltpu.VMEM((EMBED_DIM, B_PER_W), jnp.float32),
            pltpu.VMEM((48,), jnp.float32),
            pltpu.VMEM((B_PER_W,), jnp.float32),
            pltpu.SemaphoreType.DMA,
            pltpu.SemaphoreType.DMA,
        ],
    )(items, itab, stage, params)


def kernel(users, items, user_table, item_table, W, b):
    params = jnp.zeros((48,), jnp.float32)
    params = params.at[:EMBED_DIM].set(W.reshape(-1))
    params = params.at[EMBED_DIM:EMBED_DIM + 1].set(b)
    return _gmf(users, items, user_table, item_table, params)
